# baseline (speedup 1.0000x reference)
import functools

import jax
import jax.numpy as jnp
from jax import lax
from jax.experimental import pallas as pl
from jax.experimental.pallas import tpu as pltpu
from jax.experimental.pallas import tpu_sc as plsc

N_NODES = 100000
NUM_ELEMENTS = 89
NUM_CHANNEL = 128
N_SYSTEMS = 64

NUM_WORKERS = 32
PER_WORKER = 3128
N_PAD = NUM_WORKERS * PER_WORKER
CHUNK = 384
NCHUNKS = 9
NSPLIT = 4
ESTRIDE = (NUM_ELEMENTS + NSPLIT - 1) // NSPLIT


def _make_scale_kernel():
    mesh = plsc.VectorSubcoreMesh(core_axis_name="c", subcore_axis_name="s")

    @functools.partial(
        pl.kernel,
        mesh=mesh,
        compiler_params=pltpu.CompilerParams(needs_layout_passes=False),
        out_type=jax.ShapeDtypeStruct((N_PAD,), jnp.float32),
        scratch_types=[
            pltpu.VMEM((CHUNK, NUM_ELEMENTS), jnp.float32),
            pltpu.VMEM((CHUNK, NUM_ELEMENTS), jnp.float32),
            pltpu.VMEM((PER_WORKER,), jnp.int32),
            pltpu.VMEM((PER_WORKER,), jnp.float32),
            pltpu.VMEM((N_SYSTEMS,), jnp.float32),
            pltpu.VMEM((NUM_ELEMENTS,), jnp.float32),
            pltpu.SemaphoreType.DMA,
            pltpu.SemaphoreType.DMA,
        ],
    )
    def scale_kernel(attrs_hbm, batch_hbm, attr_hbm, ew_hbm, s_hbm,
                     attrs_v0, attrs_v1, batch_v, s_v, attr_v, ew_v,
                     sem0, sem1):
        wid = lax.axis_index("s") * 2 + lax.axis_index("c")
        start = wid * PER_WORKER
        end = jnp.minimum(start + PER_WORKER, N_NODES)

        pltpu.sync_copy(attr_hbm, attr_v)
        pltpu.sync_copy(ew_hbm, ew_v)
        pltpu.sync_copy(batch_hbm.at[pl.ds(start, PER_WORKER)], batch_v)

        bufs = (attrs_v0, attrs_v1)
        sems = (sem0, sem1)
        lanes = lax.iota(jnp.int32, 16)

        def chunk_base(j):
            return jnp.minimum(start + j * CHUNK, end - CHUNK)

        def start_dma(j, b):
            base = chunk_base(j)
            return pltpu.async_copy(
                attrs_hbm.at[pl.ds(base, CHUNK)], bufs[b], sems[b])

        def wait_dma(b):
            pltpu.make_async_copy(
                attrs_hbm.at[pl.ds(0, CHUNK)], bufs[b], sems[b]).wait()

        def compute_chunk(j, b):
            base = chunk_base(j)
            abuf = bufs[b]

            def group_body(g, carry):
                rows = g * 16 + lanes
                best = None
                bidx = None
                for k in range(NSPLIT):
                    xv = lanes + (k * ESTRIDE)
                    xv = jnp.where(xv >= NUM_ELEMENTS, xv - NUM_ELEMENTS, xv)
                    bk = plsc.load_gather(abuf, [rows, xv])
                    ik = xv
                    e_hi = min((k + 1) * ESTRIDE, NUM_ELEMENTS)
                    for _ in range(k * ESTRIDE + 1, e_hi):
                        xv = xv + 1
                        xv = jnp.where(xv == NUM_ELEMENTS,
                                       jnp.zeros((16,), jnp.int32), xv)
                        v = plsc.load_gather(abuf, [rows, xv])
                        upd = (v > bk) | ((v == bk) & (xv < ik))
                        bk = jnp.where(upd, v, bk)
                        ik = jnp.where(upd, xv, ik)
                    if best is None:
                        best, bidx = bk, ik
                    else:
                        upd = (bk > best) | ((bk == best) & (ik < bidx))
                        best = jnp.where(upd, bk, best)
                        bidx = jnp.where(upd, ik, bidx)
                ewv = plsc.load_gather(ew_v, [bidx])
                off = base - start
                bv = batch_v[pl.ds(off + g * 16, 16)]
                av = plsc.load_gather(attr_v, [bv])
                s_v[pl.ds(off + g * 16, 16)] = av * ewv
                return carry

            lax.fori_loop(0, CHUNK // 16, group_body, 0)

        start_dma(0, 0)
        start_dma(1, 1)

        def pair_body(i, carry):
            for b in range(2):
                j = i + b
                wait_dma(b)
                compute_chunk(j, b)

                @pl.when(j + 2 <= NCHUNKS - 1)
                def _():
                    start_dma(j + 2, b)
            return carry

        lax.fori_loop(0, (NCHUNKS - 1) // 2, lambda i, c: pair_body(2 * i, c), 0)
        wait_dma((NCHUNKS - 1) % 2)
        compute_chunk(NCHUNKS - 1, (NCHUNKS - 1) % 2)

        pltpu.sync_copy(s_v, s_hbm.at[pl.ds(start, PER_WORKER)])

    return scale_kernel


_scale_kernel = _make_scale_kernel()


_ROWS_PER_BLOCK = 2048
_N_BLOCKS = -(-N_NODES // _ROWS_PER_BLOCK)
_S_TILE = _ROWS_PER_BLOCK // NUM_CHANNEL


def _add_body(feats_ref, s_ref, cw_ref, out_ref):
    st = s_ref[...]
    jj = lax.broadcasted_iota(jnp.int32, (_ROWS_PER_BLOCK, NUM_CHANNEL), 0)
    qq = lax.broadcasted_iota(jnp.int32, (_ROWS_PER_BLOCK, NUM_CHANNEL), 1)
    jr = lax.broadcasted_iota(jnp.int32, (_ROWS_PER_BLOCK, _S_TILE), 0)
    rr = lax.broadcasted_iota(jnp.int32, (_ROWS_PER_BLOCK, _S_TILE), 1)
    rsel = jnp.equal(jr // NUM_CHANNEL, rr).astype(jnp.float32)
    z = jax.lax.dot_general(rsel, st, (((1,), (0,)), ((), ())),
                            precision=jax.lax.Precision.HIGHEST,
                            preferred_element_type=jnp.float32)
    picked = jnp.where(jnp.equal(qq, jj % NUM_CHANNEL), z, 0.0)
    s_col = jnp.sum(picked, axis=1, keepdims=True)
    out_ref[...] = feats_ref[...] + s_col * cw_ref[...]


def _dense_add(node_feats, s2d, cw):
    return pl.pallas_call(
        _add_body,
        grid=(_N_BLOCKS,),
        in_specs=[
            pl.BlockSpec((_ROWS_PER_BLOCK, NUM_CHANNEL), lambda i: (i, 0)),
            pl.BlockSpec((_S_TILE, NUM_CHANNEL), lambda i: (i, 0)),
            pl.BlockSpec((1, NUM_CHANNEL), lambda i: (0, 0)),
        ],
        out_specs=pl.BlockSpec((_ROWS_PER_BLOCK, NUM_CHANNEL), lambda i: (i, 0)),
        out_shape=jax.ShapeDtypeStruct((N_NODES, NUM_CHANNEL), jnp.float32),
        compiler_params=pltpu.CompilerParams(
            dimension_semantics=("arbitrary",)),
    )(node_feats, s2d, cw.reshape(1, NUM_CHANNEL))


def kernel(node_feats, node_attrs, batch, attr, element_weights, channel_weights):
    batch = batch.astype(jnp.int32)
    s_pad = _scale_kernel(node_attrs, batch, attr, element_weights)
    s2d = s_pad.reshape(N_PAD // NUM_CHANNEL, NUM_CHANNEL)
    return _dense_add(node_feats, s2d, channel_weights)


# speedup vs baseline: 1.0344x; 1.0344x over previous
import functools

import jax
import jax.numpy as jnp
from jax import lax
from jax.experimental import pallas as pl
from jax.experimental.pallas import tpu as pltpu
from jax.experimental.pallas import tpu_sc as plsc

N_NODES = 100000
NUM_ELEMENTS = 89
NUM_CHANNEL = 128
N_SYSTEMS = 64

NUM_WORKERS = 32
PER_WORKER = 3128
N_PAD = NUM_WORKERS * PER_WORKER
CHUNK = 256
NCHUNKS = 13
NSPLIT = 8
ESTRIDE = (NUM_ELEMENTS + NSPLIT - 1) // NSPLIT


def _make_scale_kernel():
    mesh = plsc.VectorSubcoreMesh(core_axis_name="c", subcore_axis_name="s")

    @functools.partial(
        pl.kernel,
        mesh=mesh,
        compiler_params=pltpu.CompilerParams(needs_layout_passes=False),
        out_type=jax.ShapeDtypeStruct((N_PAD,), jnp.float32),
        scratch_types=[
            pltpu.VMEM((CHUNK, NUM_ELEMENTS), jnp.float32),
            pltpu.VMEM((CHUNK, NUM_ELEMENTS), jnp.float32),
            pltpu.VMEM((PER_WORKER,), jnp.int32),
            pltpu.VMEM((PER_WORKER,), jnp.float32),
            pltpu.VMEM((N_SYSTEMS,), jnp.float32),
            pltpu.VMEM((NUM_ELEMENTS,), jnp.float32),
            pltpu.SemaphoreType.DMA,
            pltpu.SemaphoreType.DMA,
        ],
    )
    def scale_kernel(attrs_hbm, batch_hbm, attr_hbm, ew_hbm, s_hbm,
                     attrs_v0, attrs_v1, batch_v, s_v, attr_v, ew_v,
                     sem0, sem1):
        wid = lax.axis_index("s") * 2 + lax.axis_index("c")
        start = wid * PER_WORKER
        end = jnp.minimum(start + PER_WORKER, N_NODES)

        pltpu.sync_copy(attr_hbm, attr_v)
        pltpu.sync_copy(ew_hbm, ew_v)
        pltpu.sync_copy(batch_hbm.at[pl.ds(start, PER_WORKER)], batch_v)

        bufs = (attrs_v0, attrs_v1)
        sems = (sem0, sem1)
        lanes = lax.iota(jnp.int32, 16)

        def chunk_base(j):
            return jnp.minimum(start + j * CHUNK, end - CHUNK)

        def start_dma(j, b):
            base = chunk_base(j)
            return pltpu.async_copy(
                attrs_hbm.at[pl.ds(base, CHUNK)], bufs[b], sems[b])

        def wait_dma(b):
            pltpu.make_async_copy(
                attrs_hbm.at[pl.ds(0, CHUNK)], bufs[b], sems[b]).wait()

        def compute_chunk(j, b):
            base = chunk_base(j)
            abuf = bufs[b]

            def group_body(g, carry):
                rows = g * 16 + lanes
                best = None
                bidx = None
                for k in range(NSPLIT):
                    xv = lanes + (k * ESTRIDE)
                    xv = jnp.where(xv >= NUM_ELEMENTS, xv - NUM_ELEMENTS, xv)
                    bk = plsc.load_gather(abuf, [rows, xv])
                    ik = xv
                    e_hi = min((k + 1) * ESTRIDE, NUM_ELEMENTS)
                    for _ in range(k * ESTRIDE + 1, e_hi):
                        xv = xv + 1
                        xv = jnp.where(xv == NUM_ELEMENTS,
                                       jnp.zeros((16,), jnp.int32), xv)
                        v = plsc.load_gather(abuf, [rows, xv])
                        upd = (v > bk) | ((v == bk) & (xv < ik))
                        bk = jnp.where(upd, v, bk)
                        ik = jnp.where(upd, xv, ik)
                    if best is None:
                        best, bidx = bk, ik
                    else:
                        upd = (bk > best) | ((bk == best) & (ik < bidx))
                        best = jnp.where(upd, bk, best)
                        bidx = jnp.where(upd, ik, bidx)
                ewv = plsc.load_gather(ew_v, [bidx])
                off = base - start
                bv = batch_v[pl.ds(off + g * 16, 16)]
                av = plsc.load_gather(attr_v, [bv])
                s_v[pl.ds(off + g * 16, 16)] = av * ewv
                return carry

            lax.fori_loop(0, CHUNK // 16, group_body, 0)

        start_dma(0, 0)
        start_dma(1, 1)

        def pair_body(i, carry):
            for b in range(2):
                j = i + b
                wait_dma(b)
                compute_chunk(j, b)

                @pl.when(j + 2 <= NCHUNKS - 1)
                def _():
                    start_dma(j + 2, b)
            return carry

        lax.fori_loop(0, (NCHUNKS - 1) // 2, lambda i, c: pair_body(2 * i, c), 0)
        wait_dma((NCHUNKS - 1) % 2)
        compute_chunk(NCHUNKS - 1, (NCHUNKS - 1) % 2)

        pltpu.sync_copy(s_v, s_hbm.at[pl.ds(start, PER_WORKER)])

    return scale_kernel


_scale_kernel = _make_scale_kernel()


_ROWS_PER_BLOCK = 2048
_N_BLOCKS = -(-N_NODES // _ROWS_PER_BLOCK)
_S_TILE = _ROWS_PER_BLOCK // NUM_CHANNEL


def _add_body(feats_ref, s_ref, cw_ref, out_ref):
    st = s_ref[...]
    jj = lax.broadcasted_iota(jnp.int32, (_ROWS_PER_BLOCK, NUM_CHANNEL), 0)
    qq = lax.broadcasted_iota(jnp.int32, (_ROWS_PER_BLOCK, NUM_CHANNEL), 1)
    jr = lax.broadcasted_iota(jnp.int32, (_ROWS_PER_BLOCK, _S_TILE), 0)
    rr = lax.broadcasted_iota(jnp.int32, (_ROWS_PER_BLOCK, _S_TILE), 1)
    rsel = jnp.equal(jr // NUM_CHANNEL, rr).astype(jnp.float32)
    z = jax.lax.dot_general(rsel, st, (((1,), (0,)), ((), ())),
                            precision=jax.lax.Precision.HIGHEST,
                            preferred_element_type=jnp.float32)
    picked = jnp.where(jnp.equal(qq, jj % NUM_CHANNEL), z, 0.0)
    s_col = jnp.sum(picked, axis=1, keepdims=True)
    out_ref[...] = feats_ref[...] + s_col * cw_ref[...]


def _dense_add(node_feats, s2d, cw):
    return pl.pallas_call(
        _add_body,
        grid=(_N_BLOCKS,),
        in_specs=[
            pl.BlockSpec((_ROWS_PER_BLOCK, NUM_CHANNEL), lambda i: (i, 0)),
            pl.BlockSpec((_S_TILE, NUM_CHANNEL), lambda i: (i, 0)),
            pl.BlockSpec((1, NUM_CHANNEL), lambda i: (0, 0)),
        ],
        out_specs=pl.BlockSpec((_ROWS_PER_BLOCK, NUM_CHANNEL), lambda i: (i, 0)),
        out_shape=jax.ShapeDtypeStruct((N_NODES, NUM_CHANNEL), jnp.float32),
        compiler_params=pltpu.CompilerParams(
            dimension_semantics=("arbitrary",)),
    )(node_feats, s2d, cw.reshape(1, NUM_CHANNEL))


def kernel(node_feats, node_attrs, batch, attr, element_weights, channel_weights):
    batch = batch.astype(jnp.int32)
    s_pad = _scale_kernel(node_attrs, batch, attr, element_weights)
    s2d = s_pad.reshape(N_PAD // NUM_CHANNEL, NUM_CHANNEL)
    return _dense_add(node_feats, s2d, channel_weights)
